# Initial kernel scaffold; baseline (speedup 1.0000x reference)
import jax
import jax.numpy as jnp
from jax.experimental import pallas as pl


def kernel(target_logits, draft_token_ids, tree_mask, tree_draft_positions):
    raise NotImplementedError("write your pallas kernel here")


# speedup vs baseline: 2.7068x; 2.7068x over previous
import functools

import jax
import jax.numpy as jnp
from jax.experimental import pallas as pl
from jax.experimental.pallas import tpu as pltpu

_B = 32
_NODES = 15
_DRAFTS = 14
_DEPTH = 4
_WIDTH = 8
_VB = 2048


def _tree_kernel(logits_ref, drafts_ref, out_tokens_ref, path_masks_ref,
                 acc_max, acc_idx, *, vocab, nblocks):
    j = pl.program_id(0)

    @pl.when(j == 0)
    def _init():
        acc_max[...] = jnp.full((_B, _NODES, 1), -jnp.inf, dtype=jnp.float32)
        acc_idx[...] = jnp.zeros((_B, _NODES, 1), dtype=jnp.int32)

    x = logits_ref[...]
    col = jax.lax.broadcasted_iota(jnp.int32, (_B, _NODES, _VB), 2) + j * _VB
    x = jnp.where(col < vocab, x, -jnp.inf)
    blk_max = jnp.max(x, axis=-1, keepdims=True)
    cand = jnp.where(x == blk_max, col, jnp.iinfo(jnp.int32).max)
    blk_idx = jnp.min(cand, axis=-1, keepdims=True)

    better = blk_max > acc_max[...]
    acc_max[...] = jnp.where(better, blk_max, acc_max[...])
    acc_idx[...] = jnp.where(better, blk_idx, acc_idx[...])

    @pl.when(j == nblocks - 1)
    def _finish():
        idx = acc_idx[...].reshape(_B, _NODES)
        drafts = drafts_ref[...]
        idx_f = idx.astype(jnp.float32)

        nn = jax.lax.broadcasted_iota(jnp.int32, (_NODES, _DRAFTS), 0)
        dd = jax.lax.broadcasted_iota(jnp.int32, (_NODES, _DRAFTS), 1)
        gather_parent = (nn == dd // 2).astype(jnp.float32)
        sampled = jax.lax.dot_general(
            idx_f, gather_parent, (((1,), (0,)), ((), ())),
            preferred_element_type=jnp.float32)
        acc = (sampled.astype(jnp.int32) == drafts).astype(jnp.float32)

        d14 = jax.lax.broadcasted_iota(jnp.int32, (_DRAFTS, _WIDTH), 0)
        w8 = jax.lax.broadcasted_iota(jnp.int32, (_DRAFTS, _WIDTH), 1)
        m0 = (d14 == w8 // 4).astype(jnp.float32)
        m1 = (d14 == 2 + w8 // 2).astype(jnp.float32)
        m2 = (d14 == 6 + w8).astype(jnp.float32)
        dot = functools.partial(jax.lax.dot_general,
                                dimension_numbers=(((1,), (0,)), ((), ())),
                                preferred_element_type=jnp.float32)
        ta0 = dot(acc, m0)
        ta1 = dot(acc, m1)
        ta2 = dot(acc, m2)

        path_len = (ta0 + ta0 * ta1 + ta0 * ta1 * ta2).astype(jnp.int32)

        levels = jnp.max(path_len, axis=-1, keepdims=True)
        wi = jax.lax.broadcasted_iota(jnp.int32, (_B, _WIDTH), 1)
        widx = jnp.min(jnp.where(path_len == levels, wi, _WIDTH),
                       axis=-1, keepdims=True)

        ap = jnp.where(levels == 0, 0,
                       jnp.where(levels == 1, 1 + widx // 4,
                                 jnp.where(levels == 2, 3 + widx // 2,
                                           7 + widx)))

        a1 = ap + 1
        m1i = jax.lax.broadcasted_iota(jnp.int32, (_B, _DRAFTS), 1) + 2
        depth_a = ((a1 >= 2).astype(jnp.int32) + (a1 >= 4).astype(jnp.int32)
                   + (a1 >= 8).astype(jnp.int32))
        depth_m = ((m1i >= 2).astype(jnp.int32) + (m1i >= 4).astype(jnp.int32)
                   + (m1i >= 8).astype(jnp.int32))
        shift = depth_a - depth_m
        anc = jnp.right_shift(a1, jnp.maximum(shift, 0)) == m1i
        mask = jnp.logical_and(shift >= 0, anc)

        out14 = jnp.where(mask, drafts, -1)

        n15 = jax.lax.broadcasted_iota(jnp.int32, (_B, _NODES), 1)
        bonus = jnp.sum(jnp.where(n15 == ap, idx, 0), axis=-1,
                        keepdims=True)

        out_tokens_ref[:, :_DRAFTS] = out14
        out_tokens_ref[:, _DRAFTS:] = bonus
        path_masks_ref[...] = mask.astype(jnp.int32)


def kernel(target_logits, draft_token_ids, tree_mask, tree_draft_positions):
    vocab = target_logits.shape[-1]
    nblocks = pl.cdiv(vocab, _VB)
    logits = target_logits[:_B * _NODES].reshape(_B, _NODES, vocab)
    drafts = draft_token_ids.reshape(_B, _DRAFTS)

    out_tokens, path_masks_i32 = pl.pallas_call(
        functools.partial(_tree_kernel, vocab=vocab, nblocks=nblocks),
        grid=(nblocks,),
        in_specs=[
            pl.BlockSpec((_B, _NODES, _VB), lambda j: (0, 0, j)),
            pl.BlockSpec((_B, _DRAFTS), lambda j: (0, 0)),
        ],
        out_specs=[
            pl.BlockSpec((_B, _NODES), lambda j: (0, 0)),
            pl.BlockSpec((_B, _DRAFTS), lambda j: (0, 0)),
        ],
        out_shape=[
            jax.ShapeDtypeStruct((_B, _NODES), jnp.int32),
            jax.ShapeDtypeStruct((_B, _DRAFTS), jnp.int32),
        ],
        scratch_shapes=[
            pltpu.VMEM((_B, _NODES, 1), jnp.float32),
            pltpu.VMEM((_B, _NODES, 1), jnp.int32),
        ],
    )(logits, drafts)
    return out_tokens, path_masks_i32.astype(jnp.bool_)
